# baseline (speedup 1.0000x reference)
import functools

import jax
import jax.numpy as jnp
from jax import lax
from jax.experimental import pallas as pl
from jax.experimental.pallas import tpu as pltpu
from jax.experimental.pallas import tpu_sc as plsc

N = 10000
E = 320000
D = 128
G = 8
NP = 10240
NC = 2
NS = 16
NW = NC * NS
K = 128
CPW = 80
EP = NW * CPW * K
EPW = EP // NW
RPS = NP // NS
BN = 1024

_sc_mesh = plsc.VectorSubcoreMesh(core_axis_name="c", subcore_axis_name="s")
_sc_params = pltpu.CompilerParams(needs_layout_passes=False)


@functools.partial(
    pl.kernel,
    out_type=jax.ShapeDtypeStruct((NW, NP), jnp.float32),
    mesh=_sc_mesh,
    scratch_types=[
        pltpu.VMEM((EPW,), jnp.int32),
        pltpu.VMEM((EPW,), jnp.float32),
        pltpu.VMEM((NP,), jnp.float32),
    ],
    compiler_params=_sc_params,
)
def _deg_kernel(col_hbm, ew_hbm, znp_hbm, out_hbm, col_v, ew_v, deg_v):
    c = lax.axis_index("c")
    s = lax.axis_index("s")
    wid = c * NS + s
    pltpu.sync_copy(col_hbm.at[wid], col_v)
    pltpu.sync_copy(ew_hbm.at[wid], ew_v)
    pltpu.sync_copy(znp_hbm, deg_v)

    def body(i, carry):
        idx = col_v[pl.ds(i * 16, 16)]
        w = ew_v[pl.ds(i * 16, 16)]
        plsc.addupdate_scatter(deg_v, [idx], w)
        return carry

    lax.fori_loop(0, EPW // 16, body, 0)
    pltpu.sync_copy(deg_v, out_hbm.at[wid])


@functools.partial(
    pl.kernel,
    out_type=jax.ShapeDtypeStruct((NC, NP, D), jnp.float32),
    mesh=_sc_mesh,
    scratch_types=[
        pltpu.VMEM((CPW, K), jnp.int32),
        pltpu.VMEM((CPW, K), jnp.int32),
        pltpu.VMEM((EPW,), jnp.float32),
        pltpu.VMEM((K, D), jnp.float32),
        pltpu.VMEM_SHARED((NP, D), jnp.float32),
        pltpu.SemaphoreType.DMA,
    ],
    compiler_params=_sc_params,
)
def _edge_scatter_kernel(y_hbm, row_hbm, col_hbm, ew_hbm, zr_hbm, out_hbm,
                         row_v, col_v, ew_v, buf, acc, sem):
    c = lax.axis_index("c")
    s = lax.axis_index("s")
    wid = c * NS + s
    pltpu.sync_copy(row_hbm.at[wid], row_v)
    pltpu.sync_copy(col_hbm.at[wid], col_v)
    pltpu.sync_copy(ew_hbm.at[wid], ew_v)
    pltpu.sync_copy(zr_hbm, acc.at[pl.ds(s * RPS, RPS)])
    plsc.subcore_barrier()

    def chunk(g, carry):
        pltpu.async_copy(y_hbm.at[row_v.at[g]], buf, sem).wait()

        def scale16(g2, inner):
            base = g2 * 16
            wv = ew_v[pl.ds(g * K + base, 16)]
            for lane in range(16):
                w = wv[lane]
                j = base + lane
                for i in range(D // 16):
                    sl = pl.ds(i * 16, 16)
                    buf[j, sl] = buf[j, sl] * w
            return inner

        lax.fori_loop(0, K // 16, scale16, 0)
        pltpu.sync_copy(buf, acc.at[col_v.at[g]], add=True)
        return carry

    lax.fori_loop(0, CPW, chunk, 0)
    plsc.subcore_barrier()
    for k in range(RPS // K):
        base = s * RPS + k * K
        pltpu.sync_copy(acc.at[pl.ds(base, K)], buf)
        pltpu.sync_copy(buf, out_hbm.at[c, pl.ds(base, K)])


def _dinv_body(parts_ref, o_ref):
    deg = jnp.sum(parts_ref[...], axis=1, keepdims=True) + 1.0
    o_ref[...] = jnp.broadcast_to(lax.rsqrt(deg), (NP, D))


def _mm1_body(x_ref, w_ref, dinvb_ref, xw_ref, y_ref):
    xw = jnp.dot(x_ref[...], w_ref[...], preferred_element_type=jnp.float32)
    xw_ref[...] = xw
    y_ref[...] = xw * dinvb_ref[...]


def _layer_body(a0_ref, a1_ref, xw_ref, dinvb_ref, b_ref, w_ref,
                xwn_ref, yn_ref):
    dinv = dinvb_ref[...]
    h = dinv * (a0_ref[...] + a1_ref[...]) + dinv * dinv * xw_ref[...] + b_ref[...]
    h = jnp.maximum(h, 0.0)
    xwn = jnp.dot(h, w_ref[...], preferred_element_type=jnp.float32)
    xwn_ref[...] = xwn
    yn_ref[...] = xwn * dinv


def _pool_body(a0_ref, a1_ref, xw_ref, dinvb_ref, b_ref, bat_ref,
               sums_ref, cnts_ref):
    dinv = dinvb_ref[...]
    h3 = dinv * (a0_ref[...] + a1_ref[...]) + dinv * dinv * xw_ref[...] + b_ref[...]
    onehot = (bat_ref[...] ==
              lax.broadcasted_iota(jnp.int32, (BN, D), 1)).astype(jnp.float32)
    dn = (((0,), (0,)), ((), ()))
    psum = lax.dot_general(onehot, h3, dn, preferred_element_type=jnp.float32)
    pcnt = lax.dot_general(onehot, jnp.ones_like(h3), dn,
                           preferred_element_type=jnp.float32)

    @pl.when(pl.program_id(0) == 0)
    def _():
        sums_ref[...] = psum
        cnts_ref[...] = pcnt

    @pl.when(pl.program_id(0) != 0)
    def _():
        sums_ref[...] += psum
        cnts_ref[...] += pcnt


def _head_body(sums_ref, cnts_ref, w_ref, b_ref, o_ref):
    pooled = sums_ref[...] / jnp.maximum(cnts_ref[...], 1.0)
    p8 = pooled[0:G, :]
    o_ref[...] = jnp.dot(p8, w_ref[...],
                         preferred_element_type=jnp.float32) + b_ref[...]


def _row_spec():
    return pl.BlockSpec((BN, D), lambda i: (i, 0))


def _full_spec(shape):
    return pl.BlockSpec(shape, lambda i: tuple(0 for _ in shape))


def kernel(x, edge_index, edge_weight, batch,
           W1, b1, W2, b2, W3, b3, Wr, br, Wc, bc):
    f32 = jnp.float32
    row = edge_index[0]
    col = edge_index[1]

    pad_e = EP - E
    rowp = jnp.concatenate([row, jnp.zeros((pad_e,), row.dtype)])
    colp = jnp.concatenate([col, jnp.zeros((pad_e,), col.dtype)])
    ewp = jnp.concatenate([edge_weight, jnp.zeros((pad_e,), f32)])

    row3 = rowp.reshape(NW, CPW, K)
    col3 = colp.reshape(NW, CPW, K)
    colf = colp.reshape(NW, EPW)
    ewf = ewp.reshape(NW, EPW)
    x_p = jnp.concatenate([x, jnp.zeros((NP - N, D), f32)])
    z_np = jnp.zeros((NP,), f32)
    z_rows = jnp.zeros((RPS, D), f32)
    batch_p = jnp.concatenate([batch, jnp.full((NP - N,), G, batch.dtype)])
    batchb = jnp.broadcast_to(batch_p.astype(jnp.int32)[:, None], (NP, D))
    b1r = b1.reshape(1, D)
    b2r = b2.reshape(1, D)
    b3r = b3.reshape(1, D)
    w_head = jnp.zeros((D, D), f32).at[:, 0:3].set(Wr).at[:, 3:5].set(Wc)
    b_head = jnp.zeros((1, D), f32).at[0, 0:3].set(br).at[0, 3:5].set(bc)

    nb = NP // BN

    deg_parts = _deg_kernel(colf, ewf, z_np)
    dinvb = pl.pallas_call(
        _dinv_body,
        out_shape=jax.ShapeDtypeStruct((NP, D), f32),
        grid=(1,),
        in_specs=[_full_spec((NP, NW))],
        out_specs=_full_spec((NP, D)),
    )(deg_parts.T)

    xw1, y1 = pl.pallas_call(
        _mm1_body,
        out_shape=(jax.ShapeDtypeStruct((NP, D), f32),
                   jax.ShapeDtypeStruct((NP, D), f32)),
        grid=(nb,),
        in_specs=[_row_spec(), _full_spec((D, D)), _row_spec()],
        out_specs=(_row_spec(), _row_spec()),
    )(x_p, W1, dinvb)

    def tc_layer(acc, xw, b_r, w_next):
        return pl.pallas_call(
            _layer_body,
            out_shape=(jax.ShapeDtypeStruct((NP, D), f32),
                       jax.ShapeDtypeStruct((NP, D), f32)),
            grid=(nb,),
            in_specs=[_row_spec(), _row_spec(), _row_spec(), _row_spec(),
                      _full_spec((1, D)), _full_spec((D, D))],
            out_specs=(_row_spec(), _row_spec()),
        )(acc[0], acc[1], xw, dinvb, b_r, w_next)

    acc1 = _edge_scatter_kernel(y1, row3, col3, ewf, z_rows)
    xw2, y2 = tc_layer(acc1, xw1, b1r, W2)
    acc2 = _edge_scatter_kernel(y2, row3, col3, ewf, z_rows)
    xw3, y3 = tc_layer(acc2, xw2, b2r, W3)
    acc3 = _edge_scatter_kernel(y3, row3, col3, ewf, z_rows)

    sums, cnts = pl.pallas_call(
        _pool_body,
        out_shape=(jax.ShapeDtypeStruct((D, D), f32),
                   jax.ShapeDtypeStruct((D, D), f32)),
        grid=(nb,),
        in_specs=[_row_spec(), _row_spec(), _row_spec(), _row_spec(),
                  _full_spec((1, D)), _row_spec()],
        out_specs=(_full_spec((D, D)), _full_spec((D, D))),
    )(acc3[0], acc3[1], xw3, dinvb, b3r, batchb)

    out = pl.pallas_call(
        _head_body,
        out_shape=jax.ShapeDtypeStruct((G, D), f32),
        grid=(1,),
        in_specs=[_full_spec((D, D)), _full_spec((D, D)),
                  _full_spec((D, D)), _full_spec((1, D))],
        out_specs=_full_spec((G, D)),
    )(sums, cnts, w_head, b_head)

    return out[:, 0:3], out[:, 3:5]


# speedup vs baseline: 1.5248x; 1.5248x over previous
import functools

import jax
import jax.numpy as jnp
from jax import lax
from jax.experimental import pallas as pl
from jax.experimental.pallas import tpu as pltpu
from jax.experimental.pallas import tpu_sc as plsc

N = 10000
E = 320000
D = 128
G = 8
NP = 10240
NC = 2
NS = 16
NW = NC * NS
K = 128
CPW = 79
EP = NW * CPW * K
EPW = EP // NW
RPS = NP // NS
BN = 1024

_sc_mesh = plsc.VectorSubcoreMesh(core_axis_name="c", subcore_axis_name="s")
_sc_params = pltpu.CompilerParams(needs_layout_passes=False)


@functools.partial(
    pl.kernel,
    out_type=jax.ShapeDtypeStruct((NW, NP), jnp.float32),
    mesh=_sc_mesh,
    scratch_types=[
        pltpu.VMEM((EPW,), jnp.int32),
        pltpu.VMEM((EPW,), jnp.float32),
        pltpu.VMEM((NP,), jnp.float32),
    ],
    compiler_params=_sc_params,
)
def _deg_kernel(col_hbm, ew_hbm, znp_hbm, out_hbm, col_v, ew_v, deg_v):
    c = lax.axis_index("c")
    s = lax.axis_index("s")
    wid = c * NS + s
    pltpu.sync_copy(col_hbm.at[wid], col_v)
    pltpu.sync_copy(ew_hbm.at[wid], ew_v)
    pltpu.sync_copy(znp_hbm, deg_v)

    def body(i, carry):
        idx = col_v[pl.ds(i * 16, 16)]
        w = ew_v[pl.ds(i * 16, 16)]
        plsc.addupdate_scatter(deg_v, [idx], w)
        return carry

    lax.fori_loop(0, EPW // 16, body, 0)
    pltpu.sync_copy(deg_v, out_hbm.at[wid])


@functools.partial(
    pl.kernel,
    out_type=jax.ShapeDtypeStruct((NC, NP, D), jnp.float32),
    mesh=_sc_mesh,
    scratch_types=[
        pltpu.VMEM((CPW, K), jnp.int32),
        pltpu.VMEM((CPW, K), jnp.int32),
        pltpu.VMEM((EPW,), jnp.float32),
        pltpu.VMEM((K, D), jnp.float32),
        pltpu.VMEM_SHARED((NP, D), jnp.float32),
        pltpu.SemaphoreType.DMA,
    ],
    compiler_params=_sc_params,
)
def _edge_scatter_kernel(y_hbm, row_hbm, col_hbm, ew_hbm, zr_hbm, out_hbm,
                         row_v, col_v, ew_v, buf, acc, sem):
    c = lax.axis_index("c")
    s = lax.axis_index("s")
    wid = c * NS + s
    pltpu.sync_copy(row_hbm.at[wid], row_v)
    pltpu.sync_copy(col_hbm.at[wid], col_v)
    pltpu.sync_copy(ew_hbm.at[wid], ew_v)
    pltpu.sync_copy(zr_hbm, acc.at[pl.ds(s * RPS, RPS)])
    plsc.subcore_barrier()

    def chunk(g, carry):
        pltpu.async_copy(y_hbm.at[row_v.at[g]], buf, sem).wait()

        def scale16(g2, inner):
            base = g2 * 16
            wv = ew_v[pl.ds(g * K + base, 16)]
            for lane in range(16):
                w = wv[lane]
                j = base + lane
                for i in range(D // 16):
                    sl = pl.ds(i * 16, 16)
                    buf[j, sl] = buf[j, sl] * w
            return inner

        lax.fori_loop(0, K // 16, scale16, 0)
        pltpu.sync_copy(buf, acc.at[col_v.at[g]], add=True)
        return carry

    lax.fori_loop(0, CPW, chunk, 0)
    plsc.subcore_barrier()
    for k in range(RPS // K):
        base = s * RPS + k * K
        pltpu.sync_copy(acc.at[pl.ds(base, K)], buf)
        pltpu.sync_copy(buf, out_hbm.at[c, pl.ds(base, K)])


def _dinv_body(parts_ref, o_ref):
    deg = jnp.sum(parts_ref[...], axis=1, keepdims=True) + 1.0
    o_ref[...] = jnp.broadcast_to(lax.rsqrt(deg), (NP, D))


def _mm1_body(x_ref, w_ref, dinvb_ref, xw_ref, y_ref):
    xw = jnp.dot(x_ref[...], w_ref[...], preferred_element_type=jnp.float32)
    xw_ref[...] = xw
    y_ref[...] = xw * dinvb_ref[...]


def _layer_body(a0_ref, a1_ref, xw_ref, dinvb_ref, b_ref, w_ref,
                xwn_ref, yn_ref):
    dinv = dinvb_ref[...]
    h = dinv * (a0_ref[...] + a1_ref[...]) + dinv * dinv * xw_ref[...] + b_ref[...]
    h = jnp.maximum(h, 0.0)
    xwn = jnp.dot(h, w_ref[...], preferred_element_type=jnp.float32)
    xwn_ref[...] = xwn
    yn_ref[...] = xwn * dinv


def _pool_body(a0_ref, a1_ref, xw_ref, dinvb_ref, b_ref, bat_ref,
               sums_ref, cnts_ref):
    dinv = dinvb_ref[...]
    h3 = dinv * (a0_ref[...] + a1_ref[...]) + dinv * dinv * xw_ref[...] + b_ref[...]
    onehot = (bat_ref[...] ==
              lax.broadcasted_iota(jnp.int32, (BN, D), 1)).astype(jnp.float32)
    dn = (((0,), (0,)), ((), ()))
    psum = lax.dot_general(onehot, h3, dn, preferred_element_type=jnp.float32)
    pcnt = lax.dot_general(onehot, jnp.ones_like(h3), dn,
                           preferred_element_type=jnp.float32)

    @pl.when(pl.program_id(0) == 0)
    def _():
        sums_ref[...] = psum
        cnts_ref[...] = pcnt

    @pl.when(pl.program_id(0) != 0)
    def _():
        sums_ref[...] += psum
        cnts_ref[...] += pcnt


def _head_body(sums_ref, cnts_ref, w_ref, b_ref, o_ref):
    pooled = sums_ref[...] / jnp.maximum(cnts_ref[...], 1.0)
    p8 = pooled[0:G, :]
    o_ref[...] = jnp.dot(p8, w_ref[...],
                         preferred_element_type=jnp.float32) + b_ref[...]


def _row_spec():
    return pl.BlockSpec((BN, D), lambda i: (i, 0))


def _full_spec(shape):
    return pl.BlockSpec(shape, lambda i: tuple(0 for _ in shape))


def kernel(x, edge_index, edge_weight, batch,
           W1, b1, W2, b2, W3, b3, Wr, br, Wc, bc):
    f32 = jnp.float32
    row = edge_index[0]
    col = edge_index[1]

    pad_e = EP - E
    rowp = jnp.concatenate([row, jnp.zeros((pad_e,), row.dtype)])
    colp = jnp.concatenate([col, jnp.zeros((pad_e,), col.dtype)])
    ewp = jnp.concatenate([edge_weight, jnp.zeros((pad_e,), f32)])

    row3 = rowp.reshape(NW, CPW, K)
    col3 = colp.reshape(NW, CPW, K)
    colf = colp.reshape(NW, EPW)
    ewf = ewp.reshape(NW, EPW)
    x_p = jnp.concatenate([x, jnp.zeros((NP - N, D), f32)])
    z_np = jnp.zeros((NP,), f32)
    z_rows = jnp.zeros((RPS, D), f32)
    batch_p = jnp.concatenate([batch, jnp.full((NP - N,), G, batch.dtype)])
    batchb = jnp.broadcast_to(batch_p.astype(jnp.int32)[:, None], (NP, D))
    b1r = b1.reshape(1, D)
    b2r = b2.reshape(1, D)
    b3r = b3.reshape(1, D)
    w_head = jnp.zeros((D, D), f32).at[:, 0:3].set(Wr).at[:, 3:5].set(Wc)
    b_head = jnp.zeros((1, D), f32).at[0, 0:3].set(br).at[0, 3:5].set(bc)

    nb = NP // BN

    deg_parts = _deg_kernel(colf, ewf, z_np)
    dinvb = pl.pallas_call(
        _dinv_body,
        out_shape=jax.ShapeDtypeStruct((NP, D), f32),
        grid=(1,),
        in_specs=[_full_spec((NP, NW))],
        out_specs=_full_spec((NP, D)),
    )(deg_parts.T)

    xw1, y1 = pl.pallas_call(
        _mm1_body,
        out_shape=(jax.ShapeDtypeStruct((NP, D), f32),
                   jax.ShapeDtypeStruct((NP, D), f32)),
        grid=(nb,),
        in_specs=[_row_spec(), _full_spec((D, D)), _row_spec()],
        out_specs=(_row_spec(), _row_spec()),
    )(x_p, W1, dinvb)

    def tc_layer(acc, xw, b_r, w_next):
        return pl.pallas_call(
            _layer_body,
            out_shape=(jax.ShapeDtypeStruct((NP, D), f32),
                       jax.ShapeDtypeStruct((NP, D), f32)),
            grid=(nb,),
            in_specs=[_row_spec(), _row_spec(), _row_spec(), _row_spec(),
                      _full_spec((1, D)), _full_spec((D, D))],
            out_specs=(_row_spec(), _row_spec()),
        )(acc[0], acc[1], xw, dinvb, b_r, w_next)

    acc1 = _edge_scatter_kernel(y1, row3, col3, ewf, z_rows)
    xw2, y2 = tc_layer(acc1, xw1, b1r, W2)
    acc2 = _edge_scatter_kernel(y2, row3, col3, ewf, z_rows)
    xw3, y3 = tc_layer(acc2, xw2, b2r, W3)
    acc3 = _edge_scatter_kernel(y3, row3, col3, ewf, z_rows)

    sums, cnts = pl.pallas_call(
        _pool_body,
        out_shape=(jax.ShapeDtypeStruct((D, D), f32),
                   jax.ShapeDtypeStruct((D, D), f32)),
        grid=(nb,),
        in_specs=[_row_spec(), _row_spec(), _row_spec(), _row_spec(),
                  _full_spec((1, D)), _row_spec()],
        out_specs=(_full_spec((D, D)), _full_spec((D, D))),
    )(acc3[0], acc3[1], xw3, dinvb, b3r, batchb)

    out = pl.pallas_call(
        _head_body,
        out_shape=jax.ShapeDtypeStruct((G, D), f32),
        grid=(1,),
        in_specs=[_full_spec((D, D)), _full_spec((D, D)),
                  _full_spec((D, D)), _full_spec((1, D))],
        out_specs=_full_spec((G, D)),
    )(sums, cnts, w_head, b_head)

    return out[:, 0:3], out[:, 3:5]


# speedup vs baseline: 1.9961x; 1.3091x over previous
import functools

import jax
import jax.numpy as jnp
from jax import lax
from jax.experimental import pallas as pl
from jax.experimental.pallas import tpu as pltpu
from jax.experimental.pallas import tpu_sc as plsc

N = 10000
E = 320000
D = 128
G = 8
NP = 10240
NC = 2
NS = 16
NW = NC * NS
K = 128
CPW = 79
EP = NW * CPW * K
EPW = EP // NW
RPS = NP // NS
BN = 1024

_sc_mesh = plsc.VectorSubcoreMesh(core_axis_name="c", subcore_axis_name="s")
_sc_params = pltpu.CompilerParams(needs_layout_passes=False)


@functools.partial(
    pl.kernel,
    out_type=jax.ShapeDtypeStruct((NW, NP), jnp.float32),
    mesh=_sc_mesh,
    scratch_types=[
        pltpu.VMEM((EPW,), jnp.int32),
        pltpu.VMEM((EPW,), jnp.float32),
        pltpu.VMEM((NP,), jnp.float32),
    ],
    compiler_params=_sc_params,
)
def _deg_kernel(col_hbm, ew_hbm, znp_hbm, out_hbm, col_v, ew_v, deg_v):
    c = lax.axis_index("c")
    s = lax.axis_index("s")
    wid = c * NS + s
    pltpu.sync_copy(col_hbm.at[wid], col_v)
    pltpu.sync_copy(ew_hbm.at[wid], ew_v)
    pltpu.sync_copy(znp_hbm, deg_v)

    def body(i, carry):
        idx = col_v[pl.ds(i * 16, 16)]
        w = ew_v[pl.ds(i * 16, 16)]
        plsc.addupdate_scatter(deg_v, [idx], w)
        return carry

    lax.fori_loop(0, EPW // 16, body, 0)
    pltpu.sync_copy(deg_v, out_hbm.at[wid])


@functools.partial(
    pl.kernel,
    out_type=jax.ShapeDtypeStruct((NC, NP, D), jnp.float32),
    mesh=_sc_mesh,
    scratch_types=[
        pltpu.VMEM((CPW, K), jnp.int32),
        pltpu.VMEM((8, K), jnp.int32),
        pltpu.VMEM((8, K), jnp.float32),
        pltpu.VMEM((K, D), jnp.float32),
        pltpu.VMEM((K, D), jnp.float32),
        pltpu.VMEM_SHARED((NP, D), jnp.float32),
        pltpu.SemaphoreType.DMA,
        pltpu.SemaphoreType.DMA,
        pltpu.SemaphoreType.DMA,
        pltpu.SemaphoreType.DMA,
        pltpu.SemaphoreType.DMA,
        pltpu.SemaphoreType.DMA,
        pltpu.SemaphoreType.DMA,
        pltpu.SemaphoreType.DMA,
        pltpu.SemaphoreType.DMA,
        pltpu.SemaphoreType.DMA,
    ],
    compiler_params=_sc_params,
)
def _edge_scatter_kernel(y_hbm, row_hbm, col_hbm, ew_hbm, zr_hbm, out_hbm,
                         col_v, rowr, ewr, bf0, bf1, acc,
                         gs0, gs1,
                         is0, is1, is2, is3, is4, is5, is6, is7):
    c = lax.axis_index("c")
    s = lax.axis_index("s")
    wid = c * NS + s
    bufs = (bf0, bf1)
    gsem = (gs0, gs1)
    isem = (is0, is1, is2, is3, is4, is5, is6, is7)
    pltpu.sync_copy(col_hbm.at[wid], col_v)
    pltpu.sync_copy(zr_hbm, acc.at[pl.ds(s * RPS, RPS)])
    plsc.subcore_barrier()

    def idx_start(g, s8):
        pltpu.async_copy(row_hbm.at[wid, g], rowr.at[s8], isem[s8])
        pltpu.async_copy(ew_hbm.at[wid, g], ewr.at[s8], isem[s8])

    def idx_wait(s8):
        pltpu.make_async_copy(row_hbm.at[wid, 0], rowr.at[s8], isem[s8]).wait()
        pltpu.make_async_copy(ew_hbm.at[wid, 0], ewr.at[s8], isem[s8]).wait()

    def gather_start(b, s8):
        pltpu.async_copy(y_hbm.at[rowr.at[s8]], bufs[b], gsem[b])

    def gather_wait(b, s8):
        pltpu.make_async_copy(y_hbm.at[rowr.at[s8]], bufs[b], gsem[b]).wait()

    def scale(b, s8):
        buf = bufs[b]

        def scale16(g2, inner):
            base = g2 * 16
            wv = ewr[s8, pl.ds(base, 16)]
            for lane in range(16):
                w = wv[lane]
                j = base + lane
                for i in range(D // 16):
                    sl = pl.ds(i * 16, 16)
                    buf[j, sl] = buf[j, sl] * w
            return inner

        lax.fori_loop(0, K // 16, scale16, 0)

    for g in range(8):
        idx_start(g, g)
    idx_wait(0)
    gather_start(0, 0)

    def outer(g0, carry):
        for s8 in range(8):
            g = g0 * 8 + s8
            p = s8 % 2
            s8n = (s8 + 1) % 8

            @pl.when(g < CPW)
            def _():
                gather_wait(p, s8)

                @pl.when(g + 1 < CPW)
                def _():
                    idx_wait(s8n)
                    gather_start(1 - p, s8n)

                scale(p, s8)

                @pl.when(g + 8 < CPW)
                def _():
                    idx_start(g + 8, s8)

                pltpu.sync_copy(bufs[p], acc.at[col_v.at[g]], add=True)
        return carry

    lax.fori_loop(0, (CPW + 7) // 8, outer, 0)
    plsc.subcore_barrier()
    for k in range(RPS // K):
        base = s * RPS + k * K
        pltpu.sync_copy(acc.at[pl.ds(base, K)], bf0)
        pltpu.sync_copy(bf0, out_hbm.at[c, pl.ds(base, K)])


def _dinv_body(parts_ref, o_ref):
    deg = jnp.sum(parts_ref[...], axis=1, keepdims=True) + 1.0
    o_ref[...] = jnp.broadcast_to(lax.rsqrt(deg), (NP, D))


def _mm1_body(x_ref, w_ref, dinvb_ref, xw_ref, y_ref):
    xw = jnp.dot(x_ref[...], w_ref[...], preferred_element_type=jnp.float32)
    xw_ref[...] = xw
    y_ref[...] = xw * dinvb_ref[...]


def _layer_body(a0_ref, a1_ref, xw_ref, dinvb_ref, b_ref, w_ref,
                xwn_ref, yn_ref):
    dinv = dinvb_ref[...]
    h = dinv * (a0_ref[...] + a1_ref[...]) + dinv * dinv * xw_ref[...] + b_ref[...]
    h = jnp.maximum(h, 0.0)
    xwn = jnp.dot(h, w_ref[...], preferred_element_type=jnp.float32)
    xwn_ref[...] = xwn
    yn_ref[...] = xwn * dinv


def _pool_body(a0_ref, a1_ref, xw_ref, dinvb_ref, b_ref, bat_ref,
               sums_ref, cnts_ref):
    dinv = dinvb_ref[...]
    h3 = dinv * (a0_ref[...] + a1_ref[...]) + dinv * dinv * xw_ref[...] + b_ref[...]
    onehot = (bat_ref[...] ==
              lax.broadcasted_iota(jnp.int32, (BN, D), 1)).astype(jnp.float32)
    dn = (((0,), (0,)), ((), ()))
    psum = lax.dot_general(onehot, h3, dn, preferred_element_type=jnp.float32)
    pcnt = lax.dot_general(onehot, jnp.ones_like(h3), dn,
                           preferred_element_type=jnp.float32)

    @pl.when(pl.program_id(0) == 0)
    def _():
        sums_ref[...] = psum
        cnts_ref[...] = pcnt

    @pl.when(pl.program_id(0) != 0)
    def _():
        sums_ref[...] += psum
        cnts_ref[...] += pcnt


def _head_body(sums_ref, cnts_ref, w_ref, b_ref, o_ref):
    pooled = sums_ref[...] / jnp.maximum(cnts_ref[...], 1.0)
    p8 = pooled[0:G, :]
    o_ref[...] = jnp.dot(p8, w_ref[...],
                         preferred_element_type=jnp.float32) + b_ref[...]


def _row_spec():
    return pl.BlockSpec((BN, D), lambda i: (i, 0))


def _full_spec(shape):
    return pl.BlockSpec(shape, lambda i: tuple(0 for _ in shape))


def kernel(x, edge_index, edge_weight, batch,
           W1, b1, W2, b2, W3, b3, Wr, br, Wc, bc):
    f32 = jnp.float32
    row = edge_index[0]
    col = edge_index[1]

    pad_e = EP - E
    rowp = jnp.concatenate([row, jnp.zeros((pad_e,), row.dtype)])
    colp = jnp.concatenate([col, jnp.zeros((pad_e,), col.dtype)])
    ewp = jnp.concatenate([edge_weight, jnp.zeros((pad_e,), f32)])

    row3 = rowp.reshape(NW, CPW, K)
    col3 = colp.reshape(NW, CPW, K)
    ew3 = ewp.reshape(NW, CPW, K)
    colf = colp.reshape(NW, EPW)
    ewf = ewp.reshape(NW, EPW)
    x_p = jnp.concatenate([x, jnp.zeros((NP - N, D), f32)])
    z_np = jnp.zeros((NP,), f32)
    z_rows = jnp.zeros((RPS, D), f32)
    batch_p = jnp.concatenate([batch, jnp.full((NP - N,), G, batch.dtype)])
    batchb = jnp.broadcast_to(batch_p.astype(jnp.int32)[:, None], (NP, D))
    b1r = b1.reshape(1, D)
    b2r = b2.reshape(1, D)
    b3r = b3.reshape(1, D)
    w_head = jnp.zeros((D, D), f32).at[:, 0:3].set(Wr).at[:, 3:5].set(Wc)
    b_head = jnp.zeros((1, D), f32).at[0, 0:3].set(br).at[0, 3:5].set(bc)

    nb = NP // BN

    deg_parts = _deg_kernel(colf, ewf, z_np)
    dinvb = pl.pallas_call(
        _dinv_body,
        out_shape=jax.ShapeDtypeStruct((NP, D), f32),
        grid=(1,),
        in_specs=[_full_spec((NP, NW))],
        out_specs=_full_spec((NP, D)),
    )(deg_parts.T)

    xw1, y1 = pl.pallas_call(
        _mm1_body,
        out_shape=(jax.ShapeDtypeStruct((NP, D), f32),
                   jax.ShapeDtypeStruct((NP, D), f32)),
        grid=(nb,),
        in_specs=[_row_spec(), _full_spec((D, D)), _row_spec()],
        out_specs=(_row_spec(), _row_spec()),
    )(x_p, W1, dinvb)

    def tc_layer(acc, xw, b_r, w_next):
        return pl.pallas_call(
            _layer_body,
            out_shape=(jax.ShapeDtypeStruct((NP, D), f32),
                       jax.ShapeDtypeStruct((NP, D), f32)),
            grid=(nb,),
            in_specs=[_row_spec(), _row_spec(), _row_spec(), _row_spec(),
                      _full_spec((1, D)), _full_spec((D, D))],
            out_specs=(_row_spec(), _row_spec()),
        )(acc[0], acc[1], xw, dinvb, b_r, w_next)

    acc1 = _edge_scatter_kernel(y1, row3, col3, ew3, z_rows)
    xw2, y2 = tc_layer(acc1, xw1, b1r, W2)
    acc2 = _edge_scatter_kernel(y2, row3, col3, ew3, z_rows)
    xw3, y3 = tc_layer(acc2, xw2, b2r, W3)
    acc3 = _edge_scatter_kernel(y3, row3, col3, ew3, z_rows)

    sums, cnts = pl.pallas_call(
        _pool_body,
        out_shape=(jax.ShapeDtypeStruct((D, D), f32),
                   jax.ShapeDtypeStruct((D, D), f32)),
        grid=(nb,),
        in_specs=[_row_spec(), _row_spec(), _row_spec(), _row_spec(),
                  _full_spec((1, D)), _row_spec()],
        out_specs=(_full_spec((D, D)), _full_spec((D, D))),
    )(acc3[0], acc3[1], xw3, dinvb, b3r, batchb)

    out = pl.pallas_call(
        _head_body,
        out_shape=jax.ShapeDtypeStruct((G, D), f32),
        grid=(1,),
        in_specs=[_full_spec((D, D)), _full_spec((D, D)),
                  _full_spec((D, D)), _full_spec((1, D))],
        out_specs=_full_spec((G, D)),
    )(sums, cnts, w_head, b_head)

    return out[:, 0:3], out[:, 3:5]


# speedup vs baseline: 2.0057x; 1.0048x over previous
import functools

import jax
import jax.numpy as jnp
from jax import lax
from jax.experimental import pallas as pl
from jax.experimental.pallas import tpu as pltpu
from jax.experimental.pallas import tpu_sc as plsc

N = 10000
E = 320000
D = 128
G = 8
NP = 10240
NC = 2
NS = 16
NW = NC * NS
K = 128
CPW = 79
EP = NW * CPW * K
EPW = EP // NW
RPS = NP // NS
BN = 1024

_sc_mesh = plsc.VectorSubcoreMesh(core_axis_name="c", subcore_axis_name="s")
_sc_params = pltpu.CompilerParams(needs_layout_passes=False)


@functools.partial(
    pl.kernel,
    out_type=jax.ShapeDtypeStruct((NW, NP), jnp.float32),
    mesh=_sc_mesh,
    scratch_types=[
        pltpu.VMEM((EPW,), jnp.int32),
        pltpu.VMEM((EPW,), jnp.float32),
        pltpu.VMEM((NP,), jnp.float32),
    ],
    compiler_params=_sc_params,
)
def _deg_kernel(col_hbm, ew_hbm, znp_hbm, out_hbm, col_v, ew_v, deg_v):
    c = lax.axis_index("c")
    s = lax.axis_index("s")
    wid = c * NS + s
    pltpu.sync_copy(col_hbm.at[wid], col_v)
    pltpu.sync_copy(ew_hbm.at[wid], ew_v)
    pltpu.sync_copy(znp_hbm, deg_v)

    def body(i, carry):
        idx = col_v[pl.ds(i * 16, 16)]
        w = ew_v[pl.ds(i * 16, 16)]
        plsc.addupdate_scatter(deg_v, [idx], w)
        return carry

    lax.fori_loop(0, EPW // 16, body, 0)
    pltpu.sync_copy(deg_v, out_hbm.at[wid])


@functools.partial(
    pl.kernel,
    out_type=jax.ShapeDtypeStruct((NC, NP, D), jnp.float32),
    mesh=_sc_mesh,
    scratch_types=[
        pltpu.VMEM((CPW, K), jnp.int32),
        pltpu.VMEM((8, K), jnp.int32),
        pltpu.VMEM((8, K), jnp.float32),
        pltpu.VMEM((K, D), jnp.float32),
        pltpu.VMEM((K, D), jnp.float32),
        pltpu.VMEM_SHARED((NP, D), jnp.float32),
        pltpu.SemaphoreType.DMA,
        pltpu.SemaphoreType.DMA,
        pltpu.SemaphoreType.DMA,
        pltpu.SemaphoreType.DMA,
        pltpu.SemaphoreType.DMA,
        pltpu.SemaphoreType.DMA,
        pltpu.SemaphoreType.DMA,
        pltpu.SemaphoreType.DMA,
        pltpu.SemaphoreType.DMA,
        pltpu.SemaphoreType.DMA,
        pltpu.SemaphoreType.DMA,
        pltpu.SemaphoreType.DMA,
    ],
    compiler_params=_sc_params,
)
def _edge_scatter_kernel(y_hbm, row_hbm, col_hbm, ew_hbm, zr_hbm, out_hbm,
                         col_v, rowr, ewr, bf0, bf1, acc,
                         gs0, gs1, ss0, ss1,
                         is0, is1, is2, is3, is4, is5, is6, is7):
    c = lax.axis_index("c")
    s = lax.axis_index("s")
    wid = c * NS + s
    bufs = (bf0, bf1)
    gsem = (gs0, gs1)
    ssem = (ss0, ss1)
    isem = (is0, is1, is2, is3, is4, is5, is6, is7)
    pltpu.sync_copy(col_hbm.at[wid], col_v)
    pltpu.sync_copy(zr_hbm, acc.at[pl.ds(s * RPS, RPS)])
    plsc.subcore_barrier()

    def idx_start(g, s8):
        pltpu.async_copy(row_hbm.at[wid, g], rowr.at[s8], isem[s8])
        pltpu.async_copy(ew_hbm.at[wid, g], ewr.at[s8], isem[s8])

    def idx_wait(s8):
        pltpu.make_async_copy(row_hbm.at[wid, 0], rowr.at[s8], isem[s8]).wait()
        pltpu.make_async_copy(ew_hbm.at[wid, 0], ewr.at[s8], isem[s8]).wait()

    def gather_start(b, s8):
        pltpu.async_copy(y_hbm.at[rowr.at[s8]], bufs[b], gsem[b])

    def gather_wait(b, s8):
        pltpu.make_async_copy(y_hbm.at[rowr.at[s8]], bufs[b], gsem[b]).wait()

    def scatter_start(b, g):
        pltpu.async_copy(bufs[b], acc.at[col_v.at[g]], ssem[b], add=True)

    def scatter_wait(b):
        pltpu.make_async_copy(bufs[b], acc.at[col_v.at[0]], ssem[b]).wait()

    def scale(b, s8):
        buf = bufs[b]

        def scale16(g2, inner):
            base = g2 * 16
            wv = ewr[s8, pl.ds(base, 16)]
            for lane in range(16):
                w = wv[lane]
                j = base + lane
                for i in range(D // 16):
                    sl = pl.ds(i * 16, 16)
                    buf[j, sl] = buf[j, sl] * w
            return inner

        lax.fori_loop(0, K // 16, scale16, 0)

    for g in range(8):
        idx_start(g, g)
    idx_wait(0)
    gather_start(0, 0)

    def outer(g0, carry):
        for s8 in range(8):
            g = g0 * 8 + s8
            p = s8 % 2
            s8n = (s8 + 1) % 8

            @pl.when(g < CPW)
            def _():
                gather_wait(p, s8)

                @pl.when(g >= 1)
                def _():
                    scatter_wait(1 - p)

                @pl.when(g + 1 < CPW)
                def _():
                    idx_wait(s8n)
                    gather_start(1 - p, s8n)

                scale(p, s8)

                @pl.when(g + 8 < CPW)
                def _():
                    idx_start(g + 8, s8)

                scatter_start(p, g)
        return carry

    lax.fori_loop(0, (CPW + 7) // 8, outer, 0)
    scatter_wait((CPW - 1) % 2)
    plsc.subcore_barrier()
    for k in range(RPS // K):
        base = s * RPS + k * K
        pltpu.sync_copy(acc.at[pl.ds(base, K)], bf0)
        pltpu.sync_copy(bf0, out_hbm.at[c, pl.ds(base, K)])


def _dinv_body(parts_ref, o_ref):
    deg = jnp.sum(parts_ref[...], axis=1, keepdims=True) + 1.0
    o_ref[...] = jnp.broadcast_to(lax.rsqrt(deg), (NP, D))


def _mm1_body(x_ref, w_ref, dinvb_ref, xw_ref, y_ref):
    xw = jnp.dot(x_ref[...], w_ref[...], preferred_element_type=jnp.float32)
    xw_ref[...] = xw
    y_ref[...] = xw * dinvb_ref[...]


def _layer_body(a0_ref, a1_ref, xw_ref, dinvb_ref, b_ref, w_ref,
                xwn_ref, yn_ref):
    dinv = dinvb_ref[...]
    h = dinv * (a0_ref[...] + a1_ref[...]) + dinv * dinv * xw_ref[...] + b_ref[...]
    h = jnp.maximum(h, 0.0)
    xwn = jnp.dot(h, w_ref[...], preferred_element_type=jnp.float32)
    xwn_ref[...] = xwn
    yn_ref[...] = xwn * dinv


def _pool_body(a0_ref, a1_ref, xw_ref, dinvb_ref, b_ref, bat_ref,
               sums_ref, cnts_ref):
    dinv = dinvb_ref[...]
    h3 = dinv * (a0_ref[...] + a1_ref[...]) + dinv * dinv * xw_ref[...] + b_ref[...]
    onehot = (bat_ref[...] ==
              lax.broadcasted_iota(jnp.int32, (BN, D), 1)).astype(jnp.float32)
    dn = (((0,), (0,)), ((), ()))
    psum = lax.dot_general(onehot, h3, dn, preferred_element_type=jnp.float32)
    pcnt = lax.dot_general(onehot, jnp.ones_like(h3), dn,
                           preferred_element_type=jnp.float32)

    @pl.when(pl.program_id(0) == 0)
    def _():
        sums_ref[...] = psum
        cnts_ref[...] = pcnt

    @pl.when(pl.program_id(0) != 0)
    def _():
        sums_ref[...] += psum
        cnts_ref[...] += pcnt


def _head_body(sums_ref, cnts_ref, w_ref, b_ref, o_ref):
    pooled = sums_ref[...] / jnp.maximum(cnts_ref[...], 1.0)
    p8 = pooled[0:G, :]
    o_ref[...] = jnp.dot(p8, w_ref[...],
                         preferred_element_type=jnp.float32) + b_ref[...]


def _row_spec():
    return pl.BlockSpec((BN, D), lambda i: (i, 0))


def _full_spec(shape):
    return pl.BlockSpec(shape, lambda i: tuple(0 for _ in shape))


def kernel(x, edge_index, edge_weight, batch,
           W1, b1, W2, b2, W3, b3, Wr, br, Wc, bc):
    f32 = jnp.float32
    row = edge_index[0]
    col = edge_index[1]

    pad_e = EP - E
    rowp = jnp.concatenate([row, jnp.zeros((pad_e,), row.dtype)])
    colp = jnp.concatenate([col, jnp.zeros((pad_e,), col.dtype)])
    ewp = jnp.concatenate([edge_weight, jnp.zeros((pad_e,), f32)])

    row3 = rowp.reshape(NW, CPW, K)
    col3 = colp.reshape(NW, CPW, K)
    ew3 = ewp.reshape(NW, CPW, K)
    colf = colp.reshape(NW, EPW)
    ewf = ewp.reshape(NW, EPW)
    x_p = jnp.concatenate([x, jnp.zeros((NP - N, D), f32)])
    z_np = jnp.zeros((NP,), f32)
    z_rows = jnp.zeros((RPS, D), f32)
    batch_p = jnp.concatenate([batch, jnp.full((NP - N,), G, batch.dtype)])
    batchb = jnp.broadcast_to(batch_p.astype(jnp.int32)[:, None], (NP, D))
    b1r = b1.reshape(1, D)
    b2r = b2.reshape(1, D)
    b3r = b3.reshape(1, D)
    w_head = jnp.zeros((D, D), f32).at[:, 0:3].set(Wr).at[:, 3:5].set(Wc)
    b_head = jnp.zeros((1, D), f32).at[0, 0:3].set(br).at[0, 3:5].set(bc)

    nb = NP // BN

    deg_parts = _deg_kernel(colf, ewf, z_np)
    dinvb = pl.pallas_call(
        _dinv_body,
        out_shape=jax.ShapeDtypeStruct((NP, D), f32),
        grid=(1,),
        in_specs=[_full_spec((NP, NW))],
        out_specs=_full_spec((NP, D)),
    )(deg_parts.T)

    xw1, y1 = pl.pallas_call(
        _mm1_body,
        out_shape=(jax.ShapeDtypeStruct((NP, D), f32),
                   jax.ShapeDtypeStruct((NP, D), f32)),
        grid=(nb,),
        in_specs=[_row_spec(), _full_spec((D, D)), _row_spec()],
        out_specs=(_row_spec(), _row_spec()),
    )(x_p, W1, dinvb)

    def tc_layer(acc, xw, b_r, w_next):
        return pl.pallas_call(
            _layer_body,
            out_shape=(jax.ShapeDtypeStruct((NP, D), f32),
                       jax.ShapeDtypeStruct((NP, D), f32)),
            grid=(nb,),
            in_specs=[_row_spec(), _row_spec(), _row_spec(), _row_spec(),
                      _full_spec((1, D)), _full_spec((D, D))],
            out_specs=(_row_spec(), _row_spec()),
        )(acc[0], acc[1], xw, dinvb, b_r, w_next)

    acc1 = _edge_scatter_kernel(y1, row3, col3, ew3, z_rows)
    xw2, y2 = tc_layer(acc1, xw1, b1r, W2)
    acc2 = _edge_scatter_kernel(y2, row3, col3, ew3, z_rows)
    xw3, y3 = tc_layer(acc2, xw2, b2r, W3)
    acc3 = _edge_scatter_kernel(y3, row3, col3, ew3, z_rows)

    sums, cnts = pl.pallas_call(
        _pool_body,
        out_shape=(jax.ShapeDtypeStruct((D, D), f32),
                   jax.ShapeDtypeStruct((D, D), f32)),
        grid=(nb,),
        in_specs=[_row_spec(), _row_spec(), _row_spec(), _row_spec(),
                  _full_spec((1, D)), _row_spec()],
        out_specs=(_full_spec((D, D)), _full_spec((D, D))),
    )(acc3[0], acc3[1], xw3, dinvb, b3r, batchb)

    out = pl.pallas_call(
        _head_body,
        out_shape=jax.ShapeDtypeStruct((G, D), f32),
        grid=(1,),
        in_specs=[_full_spec((D, D)), _full_spec((D, D)),
                  _full_spec((D, D)), _full_spec((1, D))],
        out_specs=_full_spec((G, D)),
    )(sums, cnts, w_head, b_head)

    return out[:, 0:3], out[:, 3:5]


# speedup vs baseline: 2.0296x; 1.0119x over previous
import functools

import jax
import jax.numpy as jnp
from jax import lax
from jax.experimental import pallas as pl
from jax.experimental.pallas import tpu as pltpu
from jax.experimental.pallas import tpu_sc as plsc

N = 10000
E = 320000
D = 128
G = 8
NP = 10240
NC = 2
NS = 16
NW = NC * NS
K = 128
CPW0 = 105
CPW1 = 53
CPW = CPW0
EP = NS * (CPW0 + CPW1) * K
EPW = EP // NW
RPS = NP // NS
BN = 1024

_sc_mesh = plsc.VectorSubcoreMesh(core_axis_name="c", subcore_axis_name="s")
_sc_params = pltpu.CompilerParams(needs_layout_passes=False)


@functools.partial(
    pl.kernel,
    out_type=jax.ShapeDtypeStruct((NW, NP), jnp.float32),
    mesh=_sc_mesh,
    scratch_types=[
        pltpu.VMEM((EPW,), jnp.int32),
        pltpu.VMEM((EPW,), jnp.float32),
        pltpu.VMEM((NP,), jnp.float32),
    ],
    compiler_params=_sc_params,
)
def _deg_kernel(col_hbm, ew_hbm, znp_hbm, out_hbm, col_v, ew_v, deg_v):
    c = lax.axis_index("c")
    s = lax.axis_index("s")
    wid = c * NS + s
    pltpu.sync_copy(col_hbm.at[wid], col_v)
    pltpu.sync_copy(ew_hbm.at[wid], ew_v)
    pltpu.sync_copy(znp_hbm, deg_v)

    def body(i, carry):
        idx = col_v[pl.ds(i * 16, 16)]
        w = ew_v[pl.ds(i * 16, 16)]
        plsc.addupdate_scatter(deg_v, [idx], w)
        return carry

    lax.fori_loop(0, EPW // 16, body, 0)
    pltpu.sync_copy(deg_v, out_hbm.at[wid])


@functools.partial(
    pl.kernel,
    out_type=jax.ShapeDtypeStruct((NC, NP, D), jnp.float32),
    mesh=_sc_mesh,
    scratch_types=[
        pltpu.VMEM((CPW, K), jnp.int32),
        pltpu.VMEM((8, K), jnp.int32),
        pltpu.VMEM((8, K), jnp.float32),
        pltpu.VMEM((K, D), jnp.float32),
        pltpu.VMEM((K, D), jnp.float32),
        pltpu.VMEM_SHARED((NP, D), jnp.float32),
        pltpu.SemaphoreType.DMA,
        pltpu.SemaphoreType.DMA,
        pltpu.SemaphoreType.DMA,
        pltpu.SemaphoreType.DMA,
        pltpu.SemaphoreType.DMA,
        pltpu.SemaphoreType.DMA,
        pltpu.SemaphoreType.DMA,
        pltpu.SemaphoreType.DMA,
        pltpu.SemaphoreType.DMA,
        pltpu.SemaphoreType.DMA,
        pltpu.SemaphoreType.DMA,
        pltpu.SemaphoreType.DMA,
    ],
    compiler_params=_sc_params,
)
def _edge_scatter_kernel(y_hbm, row_hbm, col_hbm, ew_hbm, zr_hbm, out_hbm,
                         col_v, rowr, ewr, bf0, bf1, acc,
                         gs0, gs1, ss0, ss1,
                         is0, is1, is2, is3, is4, is5, is6, is7):
    c = lax.axis_index("c")
    s = lax.axis_index("s")
    wid = c * NS + s
    cpw = jnp.where(c == 0, CPW0, CPW1)
    bufs = (bf0, bf1)
    gsem = (gs0, gs1)
    ssem = (ss0, ss1)
    isem = (is0, is1, is2, is3, is4, is5, is6, is7)
    pltpu.sync_copy(col_hbm.at[wid], col_v)
    pltpu.sync_copy(zr_hbm, acc.at[pl.ds(s * RPS, RPS)])
    plsc.subcore_barrier()

    def idx_start(g, s8):
        pltpu.async_copy(row_hbm.at[wid, g], rowr.at[s8], isem[s8])
        pltpu.async_copy(ew_hbm.at[wid, g], ewr.at[s8], isem[s8])

    def idx_wait(s8):
        pltpu.make_async_copy(row_hbm.at[wid, 0], rowr.at[s8], isem[s8]).wait()
        pltpu.make_async_copy(ew_hbm.at[wid, 0], ewr.at[s8], isem[s8]).wait()

    def gather_start(b, s8):
        pltpu.async_copy(y_hbm.at[rowr.at[s8]], bufs[b], gsem[b])

    def gather_wait(b, s8):
        pltpu.make_async_copy(y_hbm.at[rowr.at[s8]], bufs[b], gsem[b]).wait()

    def scatter_start(b, g):
        pltpu.async_copy(bufs[b], acc.at[col_v.at[g]], ssem[b], add=True)

    def scatter_wait(b):
        pltpu.make_async_copy(bufs[b], acc.at[col_v.at[0]], ssem[b]).wait()

    def scale(b, s8):
        buf = bufs[b]

        def scale16(g2, inner):
            base = g2 * 16
            wv = ewr[s8, pl.ds(base, 16)]
            for lane in range(16):
                w = wv[lane]
                j = base + lane
                for i in range(D // 16):
                    sl = pl.ds(i * 16, 16)
                    buf[j, sl] = buf[j, sl] * w
            return inner

        lax.fori_loop(0, K // 16, scale16, 0)

    for g in range(8):
        idx_start(g, g)
    idx_wait(0)
    gather_start(0, 0)

    def outer(g0, carry):
        for s8 in range(8):
            g = g0 * 8 + s8
            p = s8 % 2
            s8n = (s8 + 1) % 8

            @pl.when(g < cpw)
            def _():
                gather_wait(p, s8)

                @pl.when(g >= 1)
                def _():
                    scatter_wait(1 - p)

                @pl.when(g + 1 < cpw)
                def _():
                    idx_wait(s8n)
                    gather_start(1 - p, s8n)

                scale(p, s8)

                @pl.when(g + 8 < cpw)
                def _():
                    idx_start(g + 8, s8)

                scatter_start(p, g)
        return carry

    lax.fori_loop(0, (cpw + 7) // 8, outer, 0)
    scatter_wait(0)
    plsc.subcore_barrier()
    for k in range(RPS // K):
        base = s * RPS + k * K
        pltpu.sync_copy(acc.at[pl.ds(base, K)], bf0)
        pltpu.sync_copy(bf0, out_hbm.at[c, pl.ds(base, K)])


def _dinv_body(parts_ref, o_ref):
    deg = jnp.sum(parts_ref[...], axis=1, keepdims=True) + 1.0
    o_ref[...] = jnp.broadcast_to(lax.rsqrt(deg), (NP, D))


def _mm1_body(x_ref, w_ref, dinvb_ref, xw_ref, y_ref):
    xw = jnp.dot(x_ref[...], w_ref[...], preferred_element_type=jnp.float32)
    xw_ref[...] = xw
    y_ref[...] = xw * dinvb_ref[...]


def _layer_body(a0_ref, a1_ref, xw_ref, dinvb_ref, b_ref, w_ref,
                xwn_ref, yn_ref):
    dinv = dinvb_ref[...]
    h = dinv * (a0_ref[...] + a1_ref[...]) + dinv * dinv * xw_ref[...] + b_ref[...]
    h = jnp.maximum(h, 0.0)
    xwn = jnp.dot(h, w_ref[...], preferred_element_type=jnp.float32)
    xwn_ref[...] = xwn
    yn_ref[...] = xwn * dinv


def _pool_body(a0_ref, a1_ref, xw_ref, dinvb_ref, b_ref, bat_ref,
               sums_ref, cnts_ref):
    dinv = dinvb_ref[...]
    h3 = dinv * (a0_ref[...] + a1_ref[...]) + dinv * dinv * xw_ref[...] + b_ref[...]
    onehot = (bat_ref[...] ==
              lax.broadcasted_iota(jnp.int32, (BN, D), 1)).astype(jnp.float32)
    dn = (((0,), (0,)), ((), ()))
    psum = lax.dot_general(onehot, h3, dn, preferred_element_type=jnp.float32)
    pcnt = lax.dot_general(onehot, jnp.ones_like(h3), dn,
                           preferred_element_type=jnp.float32)

    @pl.when(pl.program_id(0) == 0)
    def _():
        sums_ref[...] = psum
        cnts_ref[...] = pcnt

    @pl.when(pl.program_id(0) != 0)
    def _():
        sums_ref[...] += psum
        cnts_ref[...] += pcnt


def _head_body(sums_ref, cnts_ref, w_ref, b_ref, o_ref):
    pooled = sums_ref[...] / jnp.maximum(cnts_ref[...], 1.0)
    p8 = pooled[0:G, :]
    o_ref[...] = jnp.dot(p8, w_ref[...],
                         preferred_element_type=jnp.float32) + b_ref[...]


def _row_spec():
    return pl.BlockSpec((BN, D), lambda i: (i, 0))


def _full_spec(shape):
    return pl.BlockSpec(shape, lambda i: tuple(0 for _ in shape))


def kernel(x, edge_index, edge_weight, batch,
           W1, b1, W2, b2, W3, b3, Wr, br, Wc, bc):
    f32 = jnp.float32
    row = edge_index[0]
    col = edge_index[1]

    pad_e = EP - E
    rowp = jnp.concatenate([row, jnp.zeros((pad_e,), row.dtype)])
    colp = jnp.concatenate([col, jnp.zeros((pad_e,), col.dtype)])
    ewp = jnp.concatenate([edge_weight, jnp.zeros((pad_e,), f32)])

    def core_split(flat):
        n0 = NS * CPW0 * K
        c0 = flat[:n0].reshape(NS, CPW0, K)
        c1 = flat[n0:].reshape(NS, CPW1, K)
        c1 = jnp.concatenate(
            [c1, jnp.zeros((NS, CPW0 - CPW1, K), flat.dtype)], axis=1)
        return jnp.concatenate([c0, c1], axis=0)

    row3 = core_split(rowp)
    col3 = core_split(colp)
    ew3 = core_split(ewp)
    colf = colp.reshape(NW, EPW)
    ewf = ewp.reshape(NW, EPW)
    x_p = jnp.concatenate([x, jnp.zeros((NP - N, D), f32)])
    z_np = jnp.zeros((NP,), f32)
    z_rows = jnp.zeros((RPS, D), f32)
    batch_p = jnp.concatenate([batch, jnp.full((NP - N,), G, batch.dtype)])
    batchb = jnp.broadcast_to(batch_p.astype(jnp.int32)[:, None], (NP, D))
    b1r = b1.reshape(1, D)
    b2r = b2.reshape(1, D)
    b3r = b3.reshape(1, D)
    w_head = jnp.zeros((D, D), f32).at[:, 0:3].set(Wr).at[:, 3:5].set(Wc)
    b_head = jnp.zeros((1, D), f32).at[0, 0:3].set(br).at[0, 3:5].set(bc)

    nb = NP // BN

    deg_parts = _deg_kernel(colf, ewf, z_np)
    dinvb = pl.pallas_call(
        _dinv_body,
        out_shape=jax.ShapeDtypeStruct((NP, D), f32),
        grid=(1,),
        in_specs=[_full_spec((NP, NW))],
        out_specs=_full_spec((NP, D)),
    )(deg_parts.T)

    xw1, y1 = pl.pallas_call(
        _mm1_body,
        out_shape=(jax.ShapeDtypeStruct((NP, D), f32),
                   jax.ShapeDtypeStruct((NP, D), f32)),
        grid=(nb,),
        in_specs=[_row_spec(), _full_spec((D, D)), _row_spec()],
        out_specs=(_row_spec(), _row_spec()),
    )(x_p, W1, dinvb)

    def tc_layer(acc, xw, b_r, w_next):
        return pl.pallas_call(
            _layer_body,
            out_shape=(jax.ShapeDtypeStruct((NP, D), f32),
                       jax.ShapeDtypeStruct((NP, D), f32)),
            grid=(nb,),
            in_specs=[_row_spec(), _row_spec(), _row_spec(), _row_spec(),
                      _full_spec((1, D)), _full_spec((D, D))],
            out_specs=(_row_spec(), _row_spec()),
        )(acc[0], acc[1], xw, dinvb, b_r, w_next)

    acc1 = _edge_scatter_kernel(y1, row3, col3, ew3, z_rows)
    xw2, y2 = tc_layer(acc1, xw1, b1r, W2)
    acc2 = _edge_scatter_kernel(y2, row3, col3, ew3, z_rows)
    xw3, y3 = tc_layer(acc2, xw2, b2r, W3)
    acc3 = _edge_scatter_kernel(y3, row3, col3, ew3, z_rows)

    sums, cnts = pl.pallas_call(
        _pool_body,
        out_shape=(jax.ShapeDtypeStruct((D, D), f32),
                   jax.ShapeDtypeStruct((D, D), f32)),
        grid=(nb,),
        in_specs=[_row_spec(), _row_spec(), _row_spec(), _row_spec(),
                  _full_spec((1, D)), _row_spec()],
        out_specs=(_full_spec((D, D)), _full_spec((D, D))),
    )(acc3[0], acc3[1], xw3, dinvb, b3r, batchb)

    out = pl.pallas_call(
        _head_body,
        out_shape=jax.ShapeDtypeStruct((G, D), f32),
        grid=(1,),
        in_specs=[_full_spec((D, D)), _full_spec((D, D)),
                  _full_spec((D, D)), _full_spec((1, D))],
        out_specs=_full_spec((G, D)),
    )(sums, cnts, w_head, b_head)

    return out[:, 0:3], out[:, 3:5]
